# Initial kernel scaffold; baseline (speedup 1.0000x reference)
import jax
import jax.numpy as jnp
from jax.experimental import pallas as pl


def kernel(x, kernel):
    raise NotImplementedError("write your pallas kernel here")


# speedup vs baseline: 87.3546x; 87.3546x over previous
import functools

import jax
import jax.numpy as jnp
from jax.experimental import pallas as pl

_BIG = 3e37
_BIGX = 1e15


def _dtw_wavefront(krev_ref, xwin_ref, out_ref, *, m, ndiag):
    krev = krev_ref[...]
    big_lane = jnp.full((1, 1), _BIG, jnp.float32)
    u_init = jnp.full((1, m), _BIG, jnp.float32)
    lane = jax.lax.broadcasted_iota(jnp.int32, (1, m), 1)
    su_init = jnp.where(lane == m - 1, jnp.float32(0), _BIG)

    def body(d, carry):
        u1, su2 = carry
        xrow = xwin_ref[pl.ds(d, 1), :]
        c = (krev - xrow) ** 2
        s = jnp.concatenate([u1[:, 1:], big_lane], axis=1)
        u = c + jnp.minimum(jnp.minimum(u1, s), su2)
        return (u, s)

    u_fin, _ = jax.lax.fori_loop(0, ndiag, body, (u_init, su_init))
    out_ref[...] = u_fin[:, :1]


def kernel(x, kernel):
    m = kernel.shape[0]
    n = x.shape[0]
    ndiag = m + n - 1
    krev = kernel[::-1].reshape(1, m).astype(jnp.float32)
    pad = jnp.full((m - 1,), _BIGX, jnp.float32)
    xpad = jnp.concatenate([pad, x.astype(jnp.float32), pad])
    idx = jnp.arange(ndiag)[:, None] + jnp.arange(m)[None, :]
    xwin = jnp.take(xpad, idx, axis=0)
    out = pl.pallas_call(
        functools.partial(_dtw_wavefront, m=m, ndiag=ndiag),
        out_shape=jax.ShapeDtypeStruct((1, 1), jnp.float32),
    )(krev, xwin)
    return out[0, 0]


# speedup vs baseline: 374.0075x; 4.2815x over previous
import functools

import jax
import jax.numpy as jnp
from jax.experimental import pallas as pl

_BIG = 3e37
_BIGX = 1e15


def _dtw_wavefront(krev_ref, xwin_ref, out_ref, *, m, ndiag):
    krev = krev_ref[...]
    big_lane = jnp.full((1, 1), _BIG, jnp.float32)
    u_init = jnp.full((1, m), _BIG, jnp.float32)
    lane = jax.lax.broadcasted_iota(jnp.int32, (1, m), 1)
    su_init = jnp.where(lane == m - 1, jnp.float32(0), _BIG)

    def body(d, carry):
        u1, su2 = carry
        xrow = xwin_ref[pl.ds(d, 1), :]
        c = (krev - xrow) ** 2
        s = jnp.concatenate([u1[:, 1:], big_lane], axis=1)
        u = c + jnp.minimum(jnp.minimum(u1, s), su2)
        return (u, s)

    u_fin, _ = jax.lax.fori_loop(0, ndiag, body, (u_init, su_init), unroll=8)
    out_ref[...] = u_fin[:, :1]


def kernel(x, kernel):
    m = kernel.shape[0]
    n = x.shape[0]
    ndiag = m + n - 1
    krev = kernel[::-1].reshape(1, m).astype(jnp.float32)
    pad = jnp.full((m - 1,), _BIGX, jnp.float32)
    xpad = jnp.concatenate([pad, x.astype(jnp.float32), pad])
    xwin = jnp.stack([xpad[t : t + ndiag] for t in range(m)], axis=1)
    out = pl.pallas_call(
        functools.partial(_dtw_wavefront, m=m, ndiag=ndiag),
        out_shape=jax.ShapeDtypeStruct((1, 1), jnp.float32),
    )(krev, xwin)
    return out[0, 0]


# speedup vs baseline: 680.7659x; 1.8202x over previous
import functools

import jax
import jax.numpy as jnp
from jax.experimental import pallas as pl

_BIG = 3e37
_BIGX = 1e15


def _dtw_wavefront(krev_ref, xr_ref, out_ref, *, m, n):
    krev = krev_ref[...]
    big_lane = jnp.full((1, 1), _BIG, jnp.float32)
    u = jnp.full((1, m), _BIG, jnp.float32)
    lane = jax.lax.broadcasted_iota(jnp.int32, (1, m), 1)
    su = jnp.where(lane == m - 1, jnp.float32(0), _BIG)

    def steps(u, su, w, count):
        for t in range(count):
            c = (krev - w[:, t : t + m]) ** 2
            s = jnp.concatenate([u[:, 1:], big_lane], axis=1)
            u = c + jnp.minimum(jnp.minimum(u, su), s)
            su = s
        return u, su

    ndiag = m + n - 1
    nblocks = ndiag // m
    rem = ndiag - nblocks * m

    def block(b, carry):
        u, su = carry
        w = jnp.concatenate(
            [xr_ref[pl.ds(b, 1), :], xr_ref[pl.ds(b + 1, 1), :]], axis=1
        )
        return steps(u, su, w, m)

    u, su = jax.lax.fori_loop(0, nblocks, block, (u, su))
    if rem:
        w = jnp.concatenate([xr_ref[nblocks, :], xr_ref[nblocks + 1, :]])
        u, su = steps(u, su, w.reshape(1, 2 * m), rem)
    out_ref[...] = u[:, :1]


def kernel(x, kernel):
    m = kernel.shape[0]
    n = x.shape[0]
    krev = kernel[::-1].reshape(1, m).astype(jnp.float32)
    nrows = (n + 3 * m - 3) // m + 1
    lpad = jnp.full((m - 1,), _BIGX, jnp.float32)
    rpad = jnp.full((nrows * m - (m - 1) - n,), _BIGX, jnp.float32)
    xr = jnp.concatenate([lpad, x.astype(jnp.float32), rpad]).reshape(nrows, m)
    out = pl.pallas_call(
        functools.partial(_dtw_wavefront, m=m, n=n),
        out_shape=jax.ShapeDtypeStruct((1, 1), jnp.float32),
    )(krev, xr)
    return out[0, 0]
